# Initial kernel scaffold; baseline (speedup 1.0000x reference)
import jax
import jax.numpy as jnp
from jax.experimental import pallas as pl


def kernel(x, table):
    raise NotImplementedError("write your pallas kernel here")


# speedup vs baseline: 6.2054x; 6.2054x over previous
import functools

import jax
import jax.numpy as jnp
from jax import lax
from jax.experimental import pallas as pl
from jax.experimental.pallas import tpu as pltpu
from jax.experimental.pallas import tpu_sc as plsc

_D = 32
_NC, _NS = 2, 16
_NW = _NC * _NS
_C = 1280


@functools.partial(jax.jit, static_argnums=(2,))
def _sc_gather(idx, table_pad, total_b):
    bpw = total_b // _NW
    nchunk = bpw // _C
    mesh = plsc.VectorSubcoreMesh(core_axis_name="c", subcore_axis_name="s")

    @functools.partial(
        pl.kernel,
        out_type=jax.ShapeDtypeStruct((total_b, _D), jnp.float32),
        mesh=mesh,
        compiler_params=pltpu.CompilerParams(use_tc_tiling_on_sc=False),
        scratch_types=[
            pltpu.VMEM((_C,), jnp.int32),
            pltpu.VMEM((_C, _D), jnp.float32),
            pltpu.SemaphoreType.DMA,
        ],
    )
    def k(idx_hbm, table_hbm, out_hbm, idx_v, rows_v, sem):
        wid = lax.axis_index("s") * _NC + lax.axis_index("c")
        base = wid * bpw

        def body(g, carry):
            off = pl.multiple_of(base + g * _C, _C)
            pltpu.sync_copy(idx_hbm.at[pl.ds(off, _C)], idx_v)
            pltpu.async_copy(table_hbm.at[idx_v], rows_v, sem).wait()
            pltpu.sync_copy(rows_v, out_hbm.at[pl.ds(off, _C)])
            return carry

        lax.fori_loop(0, nchunk, body, 0)

    return k(idx, table_pad)


def kernel(x, table):
    b, h = x.shape
    total = b * h
    idx = x.reshape(total).astype(jnp.int32)
    out = _sc_gather(idx, table, total)
    return out.reshape(b, h, _D)


# speedup vs baseline: 6.4891x; 1.0457x over previous
import functools

import jax
import jax.numpy as jnp
from jax import lax
from jax.experimental import pallas as pl
from jax.experimental.pallas import tpu as pltpu
from jax.experimental.pallas import tpu_sc as plsc

_D = 32
_NC, _NS = 2, 16
_NW = _NC * _NS
_C = 1600


@functools.partial(jax.jit, static_argnums=(2,))
def _sc_gather(idx, table, total_b):
    bpw = total_b // _NW
    nchunk = bpw // _C
    assert nchunk % 2 == 0 and nchunk >= 4
    mesh = plsc.VectorSubcoreMesh(core_axis_name="c", subcore_axis_name="s")

    @functools.partial(
        pl.kernel,
        out_type=jax.ShapeDtypeStruct((total_b, _D), jnp.float32),
        mesh=mesh,
        compiler_params=pltpu.CompilerParams(use_tc_tiling_on_sc=False),
        scratch_types=[
            pltpu.VMEM((2, _C), jnp.int32),
            pltpu.VMEM((2, _C, _D), jnp.float32),
            pltpu.SemaphoreType.DMA,
            pltpu.SemaphoreType.DMA,
            pltpu.SemaphoreType.DMA,
            pltpu.SemaphoreType.DMA,
            pltpu.SemaphoreType.DMA,
            pltpu.SemaphoreType.DMA,
        ],
    )
    def k(idx_hbm, table_hbm, out_hbm, idx_v, rows_v, i0, i1, g0, g1, s0, s1):
        wid = lax.axis_index("s") * _NC + lax.axis_index("c")
        base = wid * bpw
        isems = (i0, i1)
        gsems = (g0, g1)
        ssems = (s0, s1)

        def ioff(g):
            return pl.multiple_of(base + g * _C, _C)

        def idx_load(g, b):
            pltpu.async_copy(idx_hbm.at[pl.ds(ioff(g), _C)], idx_v.at[b], isems[b])

        def idx_wait(g, b):
            pltpu.make_async_copy(idx_hbm.at[pl.ds(ioff(g), _C)], idx_v.at[b],
                                  isems[b]).wait()

        def gather_start(b):
            pltpu.async_copy(table_hbm.at[idx_v.at[b]], rows_v.at[b], gsems[b])

        def gather_wait(b):
            pltpu.make_async_copy(table_hbm.at[idx_v.at[b]], rows_v.at[b],
                                  gsems[b]).wait()

        def store_start(g, b):
            pltpu.async_copy(rows_v.at[b], out_hbm.at[pl.ds(ioff(g), _C)], ssems[b])

        def store_wait(g, b):
            pltpu.make_async_copy(rows_v.at[b], out_hbm.at[pl.ds(ioff(g), _C)],
                                  ssems[b]).wait()

        idx_load(0, 0)
        idx_load(1, 1)
        idx_wait(0, 0)
        gather_start(0)

        def body(j, carry):
            g = 2 * j

            def step(gc, b, carry):
                ob = 1 - b
                gather_wait(b)
                store_start(gc, b)
                @pl.when(gc + 1 < nchunk)
                def _():
                    idx_wait(gc + 1, ob)
                    @pl.when(gc >= 1)
                    def _():
                        store_wait(gc - 1, ob)
                    gather_start(ob)
                    @pl.when(gc + 2 < nchunk)
                    def _():
                        idx_load(gc + 2, b)
                return carry

            carry = step(g, 0, carry)
            carry = step(g + 1, 1, carry)
            return carry

        lax.fori_loop(0, nchunk // 2, body, 0)
        store_wait(nchunk - 2, 0)
        store_wait(nchunk - 1, 1)

    return k(idx, table)


def kernel(x, table):
    b, h = x.shape
    total = b * h
    idx = x.reshape(total).astype(jnp.int32)
    out = _sc_gather(idx, table, total)
    return out.reshape(b, h, _D)
